# Initial kernel scaffold; baseline (speedup 1.0000x reference)
import jax
import jax.numpy as jnp
from jax.experimental import pallas as pl


def kernel(edge_index, edge_weight, users, items, user_emb, item_emb):
    raise NotImplementedError("write your pallas kernel here")


# speedup vs baseline: 6.3388x; 6.3388x over previous
import functools

import jax
import jax.numpy as jnp
from jax import lax
from jax.experimental import pallas as pl
from jax.experimental.pallas import tpu as pltpu
from jax.experimental.pallas import tpu_sc as plsc

N_USERS = 25000
N_ITEMS = 25000
NN = N_USERS + N_ITEMS
D = 64
DH = 32
E0 = 800000
B = 16384
NLAYERS = 3

NNP = 50048
NC = 2
NS = 16
MICRO = 128
KMIC = 4
BLK = MICRO * KMIC
NBLK = 98
EPT = BLK * NBLK
EPAD = EPT * NS

RPT = NNP // NS
ZROWS = 136
BPT = B // NS


def _splat16(x):
    return jnp.broadcast_to(x, (16,))


_mesh = plsc.VectorSubcoreMesh(core_axis_name="c", subcore_axis_name="s")


@functools.partial(
    pl.kernel,
    mesh=_mesh,
    out_type=[
        jax.ShapeDtypeStruct((NLAYERS, NC * NNP, DH), jnp.float32),
        jax.ShapeDtypeStruct((NC * B, DH), jnp.float32),
        jax.ShapeDtypeStruct((NC * B, DH), jnp.float32),
    ],
    scratch_types=[
        pltpu.VMEM_SHARED((NNP, DH), jnp.float32),
        pltpu.VMEM((KMIC, MICRO), jnp.int32),
        pltpu.VMEM((KMIC, MICRO), jnp.int32),
        pltpu.VMEM((KMIC, MICRO), jnp.float32),
        pltpu.VMEM((BLK, DH), jnp.float32),
        pltpu.SemaphoreType.DMA,
    ],
    compiler_params=pltpu.CompilerParams(use_tc_tiling_on_sc=False),
)
def _lightgcn_sc(src_h, dst_h, w_h, tbl_h, users_h, items_h,
                 t_h, u_out, i_out,
                 accum, src_v, dst_v, w_v, rows_v, sem):
    c = lax.axis_index("c")
    s = lax.axis_index("s")
    coff = c * NNP

    zero16 = jnp.zeros((16,), jnp.float32)

    def _zero_accum():
        def _zfill(i, carry):
            rows_v[i, pl.ds(0, 16)] = zero16
            rows_v[i, pl.ds(16, 16)] = zero16
            return carry
        lax.fori_loop(0, ZROWS, _zfill, 0)

        def _zdma(i, carry):
            pltpu.sync_copy(rows_v.at[pl.ds(0, ZROWS)],
                            accum.at[pl.ds(s * RPT + i * ZROWS, ZROWS)])
            return carry
        lax.fori_loop(0, RPT // ZROWS, _zdma, 0)

    def _edge_blocks(tref):
        def _block(i, carry):
            blk = s * NBLK + i
            pltpu.sync_copy(src_h.at[blk], src_v)
            pltpu.sync_copy(dst_h.at[blk], dst_v)
            pltpu.sync_copy(w_h.at[blk], w_v)
            off16 = _splat16(coff)
            for j in range(KMIC):
                for k in range(MICRO // 16):
                    src_v[j, pl.ds(k * 16, 16)] = src_v[j, pl.ds(k * 16, 16)] + off16
            cps = [
                pltpu.async_copy(
                    tref.at[src_v.at[j]],
                    rows_v.at[pl.ds(j * MICRO, MICRO)],
                    sem,
                )
                for j in range(KMIC)
            ]
            for cp in cps:
                cp.wait()
            for j in range(KMIC):
                def _scale(g, carry, j=j):
                    wvec = w_v[j, pl.ds(g * 16, 16)]
                    for t in range(16):
                        r = j * MICRO + g * 16 + t
                        wspl = _splat16(wvec[t])
                        rows_v[r, pl.ds(0, 16)] = rows_v[r, pl.ds(0, 16)] * wspl
                        rows_v[r, pl.ds(16, 16)] = rows_v[r, pl.ds(16, 16)] * wspl
                    return carry
                lax.fori_loop(0, MICRO // 16, _scale, 0)
            for j in range(KMIC):
                pltpu.sync_copy(
                    rows_v.at[pl.ds(j * MICRO, MICRO)],
                    accum.at[dst_v.at[j]],
                    add=True,
                )
            return carry
        lax.fori_loop(0, NBLK, _block, 0)

    _zero_accum()
    plsc.subcore_barrier()

    for l in range(NLAYERS):
        tref = tbl_h if l == 0 else t_h.at[l - 1]
        _edge_blocks(tref)
        plsc.subcore_barrier()
        pltpu.sync_copy(
            accum.at[pl.ds(s * RPT, RPT)],
            t_h.at[l, pl.ds(coff + s * RPT, RPT)],
        )
        if l + 1 < NLAYERS:
            _zero_accum()
        plsc.subcore_barrier()

    def _batch_gather(idx_h, out_ref):
        for h in range(BPT // BLK):
            pltpu.sync_copy(idx_h.at[s * (BPT // BLK) + h], src_v)
            off16 = _splat16(coff)
            for j in range(KMIC):
                for k in range(MICRO // 16):
                    src_v[j, pl.ds(k * 16, 16)] = src_v[j, pl.ds(k * 16, 16)] + off16
            cps = [
                pltpu.async_copy(
                    tbl_h.at[src_v.at[j]],
                    rows_v.at[pl.ds(j * MICRO, MICRO)],
                    sem,
                )
                for j in range(KMIC)
            ]
            for cp in cps:
                cp.wait()
            for l in range(NLAYERS):
                for j in range(KMIC):
                    pltpu.sync_copy(
                        t_h.at[l].at[src_v.at[j]],
                        rows_v.at[pl.ds(j * MICRO, MICRO)],
                        add=True,
                    )
            quarter = jnp.full((16,), 0.25, jnp.float32)

            def _avg(r, carry):
                rows_v[r, pl.ds(0, 16)] = rows_v[r, pl.ds(0, 16)] * quarter
                rows_v[r, pl.ds(16, 16)] = rows_v[r, pl.ds(16, 16)] * quarter
                return carry

            lax.fori_loop(0, BLK, _avg, 0)
            pltpu.sync_copy(
                rows_v,
                out_ref.at[pl.ds(c * B + s * BPT + h * BLK, BLK)])

    _batch_gather(users_h, u_out)
    _batch_gather(items_h, i_out)


def kernel(edge_index, edge_weight, users, items, user_emb, item_emb):
    src = edge_index[0].astype(jnp.int32)
    dst = edge_index[1].astype(jnp.int32)
    w = edge_weight.astype(jnp.float32)
    pad = EPAD - E0
    src = jnp.concatenate([src, jnp.zeros((pad,), jnp.int32)])
    dst = jnp.concatenate([dst, jnp.zeros((pad,), jnp.int32)])
    w = jnp.concatenate([w, jnp.zeros((pad,), jnp.float32)])
    src_h = src.reshape(NS * NBLK, KMIC, MICRO)
    dst_h = dst.reshape(NS * NBLK, KMIC, MICRO)
    w_h = w.reshape(NS * NBLK, KMIC, MICRO)

    all_emb = jnp.concatenate([user_emb, item_emb], axis=0)
    rpad = jnp.zeros((NNP - NN, DH), jnp.float32)
    tbl = jnp.concatenate(
        [all_emb[:, :DH], rpad, all_emb[:, DH:], rpad], axis=0)

    users_h = users.astype(jnp.int32).reshape(NS * (BPT // BLK), KMIC, MICRO)
    items_h = (items.astype(jnp.int32) + N_USERS).reshape(NS * (BPT // BLK), KMIC, MICRO)

    t_h, u_out, i_out = _lightgcn_sc(src_h, dst_h, w_h, tbl, users_h, items_h)
    del t_h
    user_final = jnp.concatenate([u_out[:B], u_out[B:]], axis=1)
    item_final = jnp.concatenate([i_out[:B], i_out[B:]], axis=1)
    return (user_final, item_final)


# speedup vs baseline: 10.1308x; 1.5982x over previous
import functools

import jax
import jax.numpy as jnp
from jax import lax
from jax.experimental import pallas as pl
from jax.experimental.pallas import tpu as pltpu
from jax.experimental.pallas import tpu_sc as plsc

N_USERS = 25000
N_ITEMS = 25000
NN = N_USERS + N_ITEMS
D = 64
DH = 32
E0 = 800000
B = 16384
NLAYERS = 3

NNP = 50048
NC = 2
NS = 16
MICRO = 128
KMIC = 2
BLK = MICRO * KMIC
SB = 14
NSB = 14
NBLK = SB * NSB
EPT = BLK * NBLK
EPAD = EPT * NS

RPT = NNP // NS
ZROWS = 136
BPT = B // NS
BBLK = BPT // BLK


def _splat16(x):
    return jnp.broadcast_to(x, (16,))


_mesh = plsc.VectorSubcoreMesh(core_axis_name="c", subcore_axis_name="s")


@functools.partial(
    pl.kernel,
    mesh=_mesh,
    out_type=[
        jax.ShapeDtypeStruct((NLAYERS, NC * NNP, DH), jnp.float32),
        jax.ShapeDtypeStruct((B, D), jnp.float32),
        jax.ShapeDtypeStruct((B, D), jnp.float32),
    ],
    scratch_types=[
        pltpu.VMEM_SHARED((NNP, DH), jnp.float32),
        pltpu.VMEM((SB, KMIC, MICRO), jnp.int32),
        pltpu.VMEM((SB, KMIC, MICRO), jnp.int32),
        pltpu.VMEM((SB, KMIC, MICRO), jnp.float32),
        pltpu.VMEM((2, BLK, DH), jnp.float32),
        pltpu.SemaphoreType.DMA,
        pltpu.SemaphoreType.DMA,
        pltpu.SemaphoreType.DMA,
        pltpu.SemaphoreType.DMA,
    ],
    compiler_params=pltpu.CompilerParams(use_tc_tiling_on_sc=False),
)
def _lightgcn_sc(src_h, dst_h, w_h, tbl_h, users_h, items_h,
                 t_h, u_out, i_out,
                 accum, src_v, dst_v, w_v, rows_v, sem0, sem1, ssem0, ssem1):
    c = lax.axis_index("c")
    s = lax.axis_index("s")
    coff = c * NNP
    sems = (sem0, sem1)
    ssems = (ssem0, ssem1)

    zero16 = jnp.zeros((16,), jnp.float32)

    def _zero_accum():
        def _zfill(i, carry):
            rows_v[0, i, pl.ds(0, 16)] = zero16
            rows_v[0, i, pl.ds(16, 16)] = zero16
            return carry
        lax.fori_loop(0, ZROWS, _zfill, 0)

        def _zdma(i, carry):
            pltpu.sync_copy(rows_v.at[0, pl.ds(0, ZROWS)],
                            accum.at[pl.ds(s * RPT + i * ZROWS, ZROWS)])
            return carry
        lax.fori_loop(0, RPT // ZROWS, _zdma, 0)

    def _fire_gather(tref, b, p):
        for j in range(KMIC):
            pltpu.async_copy(
                tref.at[src_v.at[b, j]],
                rows_v.at[p, pl.ds(j * MICRO, MICRO)],
                sems[p],
            )

    def _drain_gather(tref, b, p):
        for j in range(KMIC):
            pltpu.make_async_copy(
                tref.at[src_v.at[b, j]],
                rows_v.at[p, pl.ds(j * MICRO, MICRO)],
                sems[p],
            ).wait()

    def _scale_rows(b, p):
        for j in range(KMIC):
            def _sc(g, carry, j=j):
                wvec = w_v[b, j, pl.ds(g * 16, 16)]
                for t in range(16):
                    r = j * MICRO + g * 16 + t
                    wspl = _splat16(wvec[t])
                    rows_v[p, r, pl.ds(0, 16)] = rows_v[p, r, pl.ds(0, 16)] * wspl
                    rows_v[p, r, pl.ds(16, 16)] = rows_v[p, r, pl.ds(16, 16)] * wspl
                return carry
            lax.fori_loop(0, MICRO // 16, _sc, 0)

    def _scatter(b, p):
        for j in range(KMIC):
            pltpu.async_copy(
                rows_v.at[p, pl.ds(j * MICRO, MICRO)],
                accum.at[dst_v.at[b, j]],
                ssems[p],
                add=True,
            )

    def _drain_scatter(p):
        for j in range(KMIC):
            pltpu.make_async_copy(
                rows_v.at[p, pl.ds(j * MICRO, MICRO)],
                accum.at[dst_v.at[0, j]],
                ssems[p],
            ).wait()

    def _edge_blocks(tref):
        def _super(sb, carry):
            base = s * NBLK + sb * SB
            pltpu.sync_copy(src_h.at[c, pl.ds(base, SB)], src_v)
            pltpu.sync_copy(dst_h.at[pl.ds(base, SB)], dst_v)
            pltpu.sync_copy(w_h.at[pl.ds(base, SB)], w_v)

            _fire_gather(tref, 0, 0)

            def _pair(q, carry):
                b0 = 2 * q
                _drain_gather(tref, b0, 0)

                @pl.when(q > 0)
                def _():
                    _drain_scatter(1)
                _fire_gather(tref, b0 + 1, 1)
                _scale_rows(b0, 0)
                _scatter(b0, 0)

                _drain_gather(tref, b0 + 1, 1)

                @pl.when(q < SB // 2 - 1)
                def _():
                    _drain_scatter(0)
                    _fire_gather(tref, b0 + 2, 0)

                _scale_rows(b0 + 1, 1)
                _scatter(b0 + 1, 1)
                return carry

            lax.fori_loop(0, SB // 2, _pair, 0)
            _drain_scatter(0)
            _drain_scatter(1)
            return carry
        lax.fori_loop(0, NSB, _super, 0)

    _zero_accum()
    plsc.subcore_barrier()

    for l in range(NLAYERS):
        tref = tbl_h if l == 0 else t_h.at[l - 1]
        _edge_blocks(tref)
        plsc.subcore_barrier()
        pltpu.sync_copy(
            accum.at[pl.ds(s * RPT, RPT)],
            t_h.at[l, pl.ds(coff + s * RPT, RPT)],
        )
        if l + 1 < NLAYERS:
            _zero_accum()
        plsc.subcore_barrier()

    def _batch_gather(idx_h, out_ref):
        for h in range(BBLK):
            pltpu.sync_copy(idx_h.at[c, s * BBLK + h], src_v.at[0])
            _fire_gather(tbl_h, 0, 0)
            _drain_gather(tbl_h, 0, 0)
            for l in range(NLAYERS):
                for j in range(KMIC):
                    pltpu.sync_copy(
                        t_h.at[l].at[src_v.at[0, j]],
                        rows_v.at[0, pl.ds(j * MICRO, MICRO)],
                        add=True,
                    )
            quarter = jnp.full((16,), 0.25, jnp.float32)

            def _avg(r, carry):
                rows_v[0, r, pl.ds(0, 16)] = rows_v[0, r, pl.ds(0, 16)] * quarter
                rows_v[0, r, pl.ds(16, 16)] = rows_v[0, r, pl.ds(16, 16)] * quarter
                return carry

            lax.fori_loop(0, BLK, _avg, 0)
            pltpu.sync_copy(
                rows_v.at[0],
                out_ref.at[pl.ds(s * BPT + h * BLK, BLK), pl.ds(c * DH, DH)])

    _batch_gather(users_h, u_out)
    _batch_gather(items_h, i_out)


def kernel(edge_index, edge_weight, users, items, user_emb, item_emb):
    src = edge_index[0].astype(jnp.int32)
    dst = edge_index[1].astype(jnp.int32)
    w = edge_weight.astype(jnp.float32)
    pad = EPAD - E0
    src = jnp.concatenate([src, jnp.zeros((pad,), jnp.int32)])
    dst = jnp.concatenate([dst, jnp.zeros((pad,), jnp.int32)])
    w = jnp.concatenate([w, jnp.zeros((pad,), jnp.float32)])
    src_r = src.reshape(NS * NBLK, KMIC, MICRO)
    src_h = jnp.stack([src_r, src_r + NNP])
    dst_h = dst.reshape(NS * NBLK, KMIC, MICRO)
    w_h = w.reshape(NS * NBLK, KMIC, MICRO)

    all_emb = jnp.concatenate([user_emb, item_emb], axis=0)
    rpad = jnp.zeros((NNP - NN, DH), jnp.float32)
    tbl = jnp.concatenate(
        [all_emb[:, :DH], rpad, all_emb[:, DH:], rpad], axis=0)

    users_r = users.astype(jnp.int32).reshape(NS * BBLK, KMIC, MICRO)
    items_r = (items.astype(jnp.int32) + N_USERS).reshape(NS * BBLK, KMIC, MICRO)
    users_h = jnp.stack([users_r, users_r + NNP])
    items_h = jnp.stack([items_r, items_r + NNP])

    t_h, u_out, i_out = _lightgcn_sc(src_h, dst_h, w_h, tbl, users_h, items_h)
    del t_h
    return (u_out, i_out)


# speedup vs baseline: 10.7527x; 1.0614x over previous
import functools

import jax
import jax.numpy as jnp
from jax import lax
from jax.experimental import pallas as pl
from jax.experimental.pallas import tpu as pltpu
from jax.experimental.pallas import tpu_sc as plsc

N_USERS = 25000
N_ITEMS = 25000
NN = N_USERS + N_ITEMS
D = 64
DH = 32
E0 = 800000
B = 16384
NLAYERS = 3

NNP = 50048
NC = 2
NS = 16
MICRO = 256
KMIC = 1
BLK = MICRO * KMIC
SB = 14
NSB = 14
NBLK = SB * NSB
EPT = BLK * NBLK
EPAD = EPT * NS

RPT = NNP // NS
ZROWS = 136
BPT = B // NS
BBLK = BPT // BLK


def _splat16(x):
    return jnp.broadcast_to(x, (16,))


_mesh = plsc.VectorSubcoreMesh(core_axis_name="c", subcore_axis_name="s")


@functools.partial(
    pl.kernel,
    mesh=_mesh,
    out_type=[
        jax.ShapeDtypeStruct((NLAYERS, NC * NNP, DH), jnp.float32),
        jax.ShapeDtypeStruct((B, D), jnp.float32),
        jax.ShapeDtypeStruct((B, D), jnp.float32),
    ],
    scratch_types=[
        pltpu.VMEM_SHARED((NNP, DH), jnp.float32),
        pltpu.VMEM((SB, KMIC, MICRO), jnp.int32),
        pltpu.VMEM((SB, KMIC, MICRO), jnp.int32),
        pltpu.VMEM((SB, KMIC, MICRO), jnp.float32),
        pltpu.VMEM((2, BLK, DH), jnp.float32),
        pltpu.SemaphoreType.DMA,
        pltpu.SemaphoreType.DMA,
        pltpu.SemaphoreType.DMA,
        pltpu.SemaphoreType.DMA,
    ],
    compiler_params=pltpu.CompilerParams(use_tc_tiling_on_sc=False),
)
def _lightgcn_sc(src_h, dst_h, w_h, tbl_h, users_h, items_h,
                 t_h, u_out, i_out,
                 accum, src_v, dst_v, w_v, rows_v, sem0, sem1, ssem0, ssem1):
    c = lax.axis_index("c")
    s = lax.axis_index("s")
    coff = c * NNP
    sems = (sem0, sem1)
    ssems = (ssem0, ssem1)

    zero16 = jnp.zeros((16,), jnp.float32)

    def _zero_accum():
        def _zfill(i, carry):
            rows_v[0, i, pl.ds(0, 16)] = zero16
            rows_v[0, i, pl.ds(16, 16)] = zero16
            return carry
        lax.fori_loop(0, ZROWS, _zfill, 0)

        def _zdma(i, carry):
            pltpu.async_copy(rows_v.at[0, pl.ds(0, ZROWS)],
                             accum.at[pl.ds(s * RPT + i * ZROWS, ZROWS)],
                             sems[0])
            return carry
        lax.fori_loop(0, RPT // ZROWS, _zdma, 0)

        def _zdrain(i, carry):
            pltpu.make_async_copy(rows_v.at[0, pl.ds(0, ZROWS)],
                                  accum.at[pl.ds(s * RPT, ZROWS)],
                                  sems[0]).wait()
            return carry
        lax.fori_loop(0, RPT // ZROWS, _zdrain, 0)

    def _fire_gather(tref, b, p):
        for j in range(KMIC):
            pltpu.async_copy(
                tref.at[src_v.at[b, j]],
                rows_v.at[p, pl.ds(j * MICRO, MICRO)],
                sems[p],
            )

    def _drain_gather(tref, b, p):
        for j in range(KMIC):
            pltpu.make_async_copy(
                tref.at[src_v.at[b, j]],
                rows_v.at[p, pl.ds(j * MICRO, MICRO)],
                sems[p],
            ).wait()

    def _scale_rows(b, p):
        for j in range(KMIC):
            def _sc(g, carry, j=j):
                wvec = w_v[b, j, pl.ds(g * 16, 16)]
                for t in range(16):
                    r = j * MICRO + g * 16 + t
                    wspl = _splat16(wvec[t])
                    rows_v[p, r, pl.ds(0, 16)] = rows_v[p, r, pl.ds(0, 16)] * wspl
                    rows_v[p, r, pl.ds(16, 16)] = rows_v[p, r, pl.ds(16, 16)] * wspl
                return carry
            lax.fori_loop(0, MICRO // 16, _sc, 0)

    def _scatter(b, p):
        for j in range(KMIC):
            pltpu.async_copy(
                rows_v.at[p, pl.ds(j * MICRO, MICRO)],
                accum.at[dst_v.at[b, j]],
                ssems[p],
                add=True,
            )

    def _drain_scatter(p):
        for j in range(KMIC):
            pltpu.make_async_copy(
                rows_v.at[p, pl.ds(j * MICRO, MICRO)],
                accum.at[dst_v.at[0, j]],
                ssems[p],
            ).wait()

    def _edge_blocks(tref):
        def _super(sb, carry):
            base = s * NBLK + sb * SB
            pltpu.async_copy(src_h.at[c, pl.ds(base, SB)], src_v, sems[0])
            pltpu.async_copy(dst_h.at[pl.ds(base, SB)], dst_v, sems[0])
            pltpu.async_copy(w_h.at[pl.ds(base, SB)], w_v, sems[0])
            pltpu.make_async_copy(src_h.at[c, pl.ds(base, SB)], src_v, sems[0]).wait()
            pltpu.make_async_copy(dst_h.at[pl.ds(base, SB)], dst_v, sems[0]).wait()
            pltpu.make_async_copy(w_h.at[pl.ds(base, SB)], w_v, sems[0]).wait()

            _fire_gather(tref, 0, 0)

            def _pair(q, carry):
                b0 = 2 * q
                _drain_gather(tref, b0, 0)

                @pl.when(q > 0)
                def _():
                    _drain_scatter(1)
                _fire_gather(tref, b0 + 1, 1)
                _scale_rows(b0, 0)
                _scatter(b0, 0)

                _drain_gather(tref, b0 + 1, 1)

                @pl.when(q < SB // 2 - 1)
                def _():
                    _drain_scatter(0)
                    _fire_gather(tref, b0 + 2, 0)

                _scale_rows(b0 + 1, 1)
                _scatter(b0 + 1, 1)
                return carry

            lax.fori_loop(0, SB // 2, _pair, 0)
            _drain_scatter(0)
            _drain_scatter(1)
            return carry
        lax.fori_loop(0, NSB, _super, 0)

    _zero_accum()
    plsc.subcore_barrier()

    for l in range(NLAYERS):
        tref = tbl_h if l == 0 else t_h.at[l - 1]
        _edge_blocks(tref)
        plsc.subcore_barrier()
        pltpu.sync_copy(
            accum.at[pl.ds(s * RPT, RPT)],
            t_h.at[l, pl.ds(coff + s * RPT, RPT)],
        )
        if l + 1 < NLAYERS:
            _zero_accum()
        plsc.subcore_barrier()

    def _batch_gather(idx_h, out_ref):
        for h in range(BBLK):
            pltpu.sync_copy(idx_h.at[c, s * BBLK + h], src_v.at[0])
            _fire_gather(tbl_h, 0, 0)
            _drain_gather(tbl_h, 0, 0)
            for l in range(NLAYERS):
                for j in range(KMIC):
                    pltpu.sync_copy(
                        t_h.at[l].at[src_v.at[0, j]],
                        rows_v.at[0, pl.ds(j * MICRO, MICRO)],
                        add=True,
                    )
            quarter = jnp.full((16,), 0.25, jnp.float32)

            def _avg(r, carry):
                rows_v[0, r, pl.ds(0, 16)] = rows_v[0, r, pl.ds(0, 16)] * quarter
                rows_v[0, r, pl.ds(16, 16)] = rows_v[0, r, pl.ds(16, 16)] * quarter
                return carry

            lax.fori_loop(0, BLK, _avg, 0)
            pltpu.sync_copy(
                rows_v.at[0],
                out_ref.at[pl.ds(s * BPT + h * BLK, BLK), pl.ds(c * DH, DH)])

    _batch_gather(users_h, u_out)
    _batch_gather(items_h, i_out)


def kernel(edge_index, edge_weight, users, items, user_emb, item_emb):
    src = edge_index[0].astype(jnp.int32)
    dst = edge_index[1].astype(jnp.int32)
    w = edge_weight.astype(jnp.float32)
    pad = EPAD - E0
    src = jnp.concatenate([src, jnp.zeros((pad,), jnp.int32)])
    dst = jnp.concatenate([dst, jnp.zeros((pad,), jnp.int32)])
    w = jnp.concatenate([w, jnp.zeros((pad,), jnp.float32)])
    src_r = src.reshape(NS * NBLK, KMIC, MICRO)
    src_h = jnp.stack([src_r, src_r + NNP])
    dst_h = dst.reshape(NS * NBLK, KMIC, MICRO)
    w_h = w.reshape(NS * NBLK, KMIC, MICRO)

    all_emb = jnp.concatenate([user_emb, item_emb], axis=0)
    rpad = jnp.zeros((NNP - NN, DH), jnp.float32)
    tbl = jnp.concatenate(
        [all_emb[:, :DH], rpad, all_emb[:, DH:], rpad], axis=0)

    users_r = users.astype(jnp.int32).reshape(NS * BBLK, KMIC, MICRO)
    items_r = (items.astype(jnp.int32) + N_USERS).reshape(NS * BBLK, KMIC, MICRO)
    users_h = jnp.stack([users_r, users_r + NNP])
    items_h = jnp.stack([items_r, items_r + NNP])

    t_h, u_out, i_out = _lightgcn_sc(src_h, dst_h, w_h, tbl, users_h, items_h)
    del t_h
    return (u_out, i_out)
